# baseline (speedup 1.0000x reference)
import functools

import jax
import jax.numpy as jnp
from jax import lax
from jax.experimental import pallas as pl
from jax.experimental.pallas import tpu as pltpu
from jax.experimental.pallas import tpu_sc as plsc

VOCAB = 100000
HID = 768
L = 16
NH = HID // L
NC = 2
NS = 16
NW = NC * NS
B = 4
S = 2048
PPW = S // NW
C = PPW
EPS = 1e-12


def _splat(ref, t):
    return plsc.load_gather(ref, [jnp.full((L,), t, jnp.int32)])


def _rsqrt(xv):
    yi = lax.bitcast_convert_type(xv, jnp.int32)
    y = lax.bitcast_convert_type(
        jnp.int32(0x5F3759DF) - lax.shift_right_logical(yi, 1), jnp.float32)
    for _ in range(3):
        y = y * (1.5 - 0.5 * xv * y * y)
    return y


mesh = plsc.VectorSubcoreMesh(core_axis_name="c", subcore_axis_name="s")


@functools.partial(
    pl.kernel,
    mesh=mesh,
    out_type=jax.ShapeDtypeStruct((B * S, HID), jnp.float32),
    compiler_params=pltpu.CompilerParams(needs_layout_passes=False),
    scratch_types=[
        pltpu.VMEM((C,), jnp.int32),
        pltpu.VMEM((C,), jnp.int32),
        pltpu.VMEM((C,), jnp.float32),
        pltpu.VMEM((C, HID), jnp.float32),
        pltpu.VMEM((PPW, HID), jnp.float32),
        pltpu.VMEM((2, HID), jnp.float32),
        pltpu.VMEM((HID,), jnp.float32),
        pltpu.VMEM((HID,), jnp.float32),
        pltpu.SemaphoreType.DMA,
    ],
)
def _emb_kernel(ids_h, tts_h, we_h, pe_h, te_h, g_h, bt_h, out_h,
                idx_v, tt_v, ttf_v, rows_v, pe_v, te_v, g_v, b_v, sem):
    wid = lax.axis_index("s") * NC + lax.axis_index("c")
    pbase = wid * PPW

    pltpu.sync_copy(pe_h.at[pl.ds(pbase, PPW)], pe_v)
    pltpu.sync_copy(te_h, te_v)
    pltpu.sync_copy(g_h, g_v)
    pltpu.sync_copy(bt_h, b_v)

    for h in range(NH):
        hs = pl.ds(h * L, L)
        te_v[1, hs] = te_v[1, hs] - te_v[0, hs]

    def fold_body(t, _):
        for h in range(NH):
            hs = pl.ds(h * L, L)
            pe_v[t, hs] = pe_v[t, hs] + te_v[0, hs]
        return 0
    lax.fori_loop(0, PPW, fold_body, 0)

    inv_h = jnp.float32(1.0 / HID)

    for b in range(B):
        tbase = b * S + pbase
        pltpu.sync_copy(ids_h.at[pl.ds(tbase, C)], idx_v)
        pltpu.sync_copy(tts_h.at[pl.ds(tbase, C)], tt_v)
        pltpu.async_copy(we_h.at[idx_v], rows_v, sem).wait()

        for q in range(C // L):
            qs = pl.ds(q * L, L)
            ttf_v[qs] = tt_v[qs].astype(jnp.float32)

        def token_body(t, _):
            fgroup = ttf_v[pl.ds((t // L) * L, L)]
            fv = fgroup.at[jnp.full((L,), t % L, jnp.int32)].get(
                mode="promise_in_bounds")
            s1 = jnp.zeros((L,), jnp.float32)
            s2 = jnp.zeros((L,), jnp.float32)
            for h in range(NH):
                hs = pl.ds(h * L, L)
                e = rows_v[t, hs] + pe_v[t, hs] + fv * te_v[1, hs]
                rows_v[t, hs] = e
                s1 = s1 + e
                s2 = s2 + e * e
            tot1 = jnp.full((L,), jnp.sum(s1))
            tot2 = jnp.full((L,), jnp.sum(s2))
            mean = tot1 * inv_h
            var = tot2 * inv_h - mean * mean
            rstd = _rsqrt(var + jnp.float32(EPS))
            for h in range(NH):
                hs = pl.ds(h * L, L)
                e = rows_v[t, hs]
                rows_v[t, hs] = (e - mean) * rstd * g_v[hs] + b_v[hs]
            return 0

        lax.fori_loop(0, C, token_body, 0)
        pltpu.sync_copy(rows_v, out_h.at[pl.ds(tbase, C)])


def kernel(input_ids, token_type_ids, word_emb, pos_emb, type_emb, gamma, beta):
    ids = input_ids.reshape(-1).astype(jnp.int32)
    tts = token_type_ids.reshape(-1).astype(jnp.int32)
    out = _emb_kernel(ids, tts, word_emb, pos_emb, type_emb, gamma, beta)
    return out.reshape(B, S, HID)


# speedup vs baseline: 2.4614x; 2.4614x over previous
import functools

import jax
import jax.numpy as jnp
from jax import lax
from jax.experimental import pallas as pl
from jax.experimental.pallas import tpu as pltpu
from jax.experimental.pallas import tpu_sc as plsc

HID = 768
L = 16
NH = HID // L
NC = 2
NS = 16
NW = NC * NS
B = 4
S = 2048
PPW = S // NW
C = 16
CPB = PPW // C
NCHUNK = B * CPB
TB = 4
EPS = 1e-12


def _rsqrt(xv):
    yi = lax.bitcast_convert_type(xv, jnp.int32)
    y = lax.bitcast_convert_type(
        jnp.int32(0x5F3759DF) - lax.shift_right_logical(yi, 1), jnp.float32)
    for _ in range(3):
        y = y * (1.5 - 0.5 * xv * y * y)
    return y


def _lane_sum(v):
    for sh in (8, 4, 2, 1):
        perm = lax.broadcasted_iota(jnp.int32, (L,), 0) ^ sh
        v = v + v.at[perm].get(mode="promise_in_bounds")
    return v


mesh = plsc.VectorSubcoreMesh(core_axis_name="c", subcore_axis_name="s")


@functools.partial(
    pl.kernel,
    mesh=mesh,
    out_type=jax.ShapeDtypeStruct((B * S, HID), jnp.float32),
    compiler_params=pltpu.CompilerParams(needs_layout_passes=False),
    scratch_types=[
        pltpu.VMEM((C,), jnp.int32),
        pltpu.VMEM((C,), jnp.int32),
        pltpu.VMEM((C,), jnp.int32),
        pltpu.VMEM((C,), jnp.float32),
        pltpu.VMEM((C, HID), jnp.float32),
        pltpu.VMEM((C, HID), jnp.float32),
        pltpu.VMEM((C, HID), jnp.float32),
        pltpu.VMEM((C, HID), jnp.float32),
        pltpu.VMEM((C, HID), jnp.float32),
        pltpu.VMEM((PPW, HID), jnp.float32),
        pltpu.VMEM((2, HID), jnp.float32),
        pltpu.VMEM((HID,), jnp.float32),
        pltpu.VMEM((HID,), jnp.float32),
        pltpu.SemaphoreType.DMA,
        pltpu.SemaphoreType.DMA,
        pltpu.SemaphoreType.DMA,
        pltpu.SemaphoreType.DMA,
    ],
)
def _emb_kernel(ids_h, tts_h, we_h, pe_h, te_h, g_h, bt_h, out_h,
                idx0, idx1, tt_v, ttf_v, gb0, gb1, ebuf_v, ob0, ob1,
                pe_v, te_v, g_v, b_v, sg0, sg1, so0, so1):
    wid = lax.axis_index("s") * NC + lax.axis_index("c")
    pbase = wid * PPW
    idxs = (idx0, idx1)
    gbs = (gb0, gb1)
    obs = (ob0, ob1)
    sgs = (sg0, sg1)
    sos = (so0, so1)

    def tbase_of(k):
        b = k // CPB
        half = k % CPB
        return b * S + pbase + half * C, half * C

    pltpu.sync_copy(pe_h.at[pl.ds(pbase, PPW)], pe_v)
    pltpu.sync_copy(te_h, te_v)
    pltpu.sync_copy(g_h, g_v)
    pltpu.sync_copy(bt_h, b_v)

    for h in range(NH):
        hs = pl.ds(h * L, L)
        te_v[1, hs] = te_v[1, hs] - te_v[0, hs]

    @plsc.parallel_loop(0, PPW, unroll=2)
    def fold_body(t):
        for h in range(NH):
            hs = pl.ds(h * L, L)
            pe_v[t, hs] = pe_v[t, hs] + te_v[0, hs]

    inv_h = jnp.float32(1.0 / HID)

    t0base, _ = tbase_of(0)
    pltpu.sync_copy(ids_h.at[pl.ds(t0base, C)], idx0)
    pltpu.async_copy(we_h.at[idx0], gb0, sg0)

    def pair_body(kk, _):
        for par in range(2):
            k = kk * 2 + par
            tbase, poff = tbase_of(k)
            nxt = 1 - par

            @pl.when(k < NCHUNK - 1)
            def _():
                ntbase, _ = tbase_of(k + 1)
                pltpu.sync_copy(ids_h.at[pl.ds(ntbase, C)], idxs[nxt])
                pltpu.async_copy(we_h.at[idxs[nxt]], gbs[nxt], sgs[nxt])

            pltpu.make_async_copy(we_h.at[idxs[par]], gbs[par],
                                  sgs[par]).wait()

            pltpu.sync_copy(tts_h.at[pl.ds(tbase, C)], tt_v)
            ttf_v[pl.ds(0, L)] = tt_v[pl.ds(0, L)].astype(jnp.float32)
            rows_v = gbs[par]
            out_v = obs[par]

            def block_body(tb, _):
                t0 = tb * TB
                fgroup = ttf_v[pl.ds(0, L)]
                lane0 = t0
                fv = [
                    fgroup.at[jnp.full((L,), lane0 + j, jnp.int32)].get(
                        mode="promise_in_bounds")
                    for j in range(TB)
                ]
                zero = jnp.zeros((L,), jnp.float32)

                @plsc.parallel_loop(0, NH, carry=tuple([zero] * (2 * TB)))
                def moments(h, acc):
                    hs = pl.ds(h * L, L)
                    d = te_v[1, hs]
                    out = []
                    for j in range(TB):
                        e = (rows_v[t0 + j, hs] + pe_v[poff + t0 + j, hs]
                             + fv[j] * d)
                        ebuf_v[t0 + j, hs] = e
                        out.append(acc[2 * j] + e)
                        out.append(acc[2 * j + 1] + e * e)
                    return tuple(out)

                aa = []
                bb = []
                for j in range(TB):
                    mean = _lane_sum(moments[2 * j]) * inv_h
                    var = (_lane_sum(moments[2 * j + 1]) * inv_h
                           - mean * mean)
                    rstd = _rsqrt(var + jnp.float32(EPS))
                    aa.append(rstd)
                    bb.append(-mean * rstd)

                @plsc.parallel_loop(0, NH)
                def normalize(h):
                    hs = pl.ds(h * L, L)
                    g = g_v[hs]
                    bt = b_v[hs]
                    for j in range(TB):
                        e = ebuf_v[t0 + j, hs]
                        out_v[t0 + j, hs] = (e * aa[j] + bb[j]) * g + bt
                return 0

            @pl.when(k >= 2)
            def _():
                ptbase, _ = tbase_of(k - 2)
                pltpu.make_async_copy(obs[par],
                                      out_h.at[pl.ds(ptbase, C)],
                                      sos[par]).wait()

            lax.fori_loop(0, C // TB, block_body, 0)
            pltpu.async_copy(obs[par], out_h.at[pl.ds(tbase, C)], sos[par])
        return 0

    lax.fori_loop(0, NCHUNK // 2, pair_body, 0)

    for k in (NCHUNK - 2, NCHUNK - 1):
        par = k % 2
        tbase, _ = tbase_of(k)
        pltpu.make_async_copy(obs[par], out_h.at[pl.ds(tbase, C)],
                              sos[par]).wait()


def kernel(input_ids, token_type_ids, word_emb, pos_emb, type_emb, gamma, beta):
    ids = input_ids.reshape(-1).astype(jnp.int32)
    tts = token_type_ids.reshape(-1).astype(jnp.int32)
    out = _emb_kernel(ids, tts, word_emb, pos_emb, type_emb, gamma, beta)
    return out.reshape(B, S, HID)


# speedup vs baseline: 2.7746x; 1.1272x over previous
import functools

import jax
import jax.numpy as jnp
from jax import lax
from jax.experimental import pallas as pl
from jax.experimental.pallas import tpu as pltpu
from jax.experimental.pallas import tpu_sc as plsc

HID = 768
L = 16
NH = HID // L
NC = 2
NS = 16
NW = NC * NS
B = 4
S = 2048
PPW = S // NW
C = 16
CPB = PPW // C
NCHUNK = B * CPB
TB = 4
EPS = 1e-12


def _rsqrt(xv):
    yi = lax.bitcast_convert_type(xv, jnp.int32)
    y = lax.bitcast_convert_type(
        jnp.int32(0x5F3759DF) - lax.shift_right_logical(yi, 1), jnp.float32)
    for _ in range(3):
        y = y * (1.5 - 0.5 * xv * y * y)
    return y


def _lane_sum(v):
    for sh in (8, 4, 2, 1):
        perm = lax.broadcasted_iota(jnp.int32, (L,), 0) ^ sh
        v = v + v.at[perm].get(mode="promise_in_bounds")
    return v


mesh = plsc.VectorSubcoreMesh(core_axis_name="c", subcore_axis_name="s")


@functools.partial(
    pl.kernel,
    mesh=mesh,
    out_type=jax.ShapeDtypeStruct((B * S, HID), jnp.float32),
    compiler_params=pltpu.CompilerParams(needs_layout_passes=False),
    scratch_types=[
        pltpu.VMEM((B * PPW,), jnp.int32),
        pltpu.VMEM((B * PPW,), jnp.int32),
        pltpu.VMEM((B * PPW,), jnp.float32),
        pltpu.VMEM((C, HID), jnp.float32),
        pltpu.VMEM((C, HID), jnp.float32),
        pltpu.VMEM((C, HID), jnp.float32),
        pltpu.VMEM((C, HID), jnp.float32),
        pltpu.VMEM((C, HID), jnp.float32),
        pltpu.VMEM((PPW, HID), jnp.float32),
        pltpu.VMEM((2, HID), jnp.float32),
        pltpu.VMEM((HID,), jnp.float32),
        pltpu.VMEM((HID,), jnp.float32),
        pltpu.SemaphoreType.DMA,
        pltpu.SemaphoreType.DMA,
        pltpu.SemaphoreType.DMA,
        pltpu.SemaphoreType.DMA,
    ],
)
def _emb_kernel(ids_h, tts_h, we_h, pe_h, te_h, g_h, bt_h, out_h,
                idsw_v, ttw_v, ttfw_v, gb0, gb1, ebuf_v, ob0, ob1,
                pe_v, te_v, g_v, b_v, sg0, sg1, so0, so1):
    wid = lax.axis_index("s") * NC + lax.axis_index("c")
    pbase = wid * PPW
    gbs = (gb0, gb1)
    obs = (ob0, ob1)
    sgs = (sg0, sg1)
    sos = (so0, so1)

    def tbase_of(k):
        b = k // CPB
        half = k % CPB
        return b * S + pbase + half * C, half * C

    pltpu.sync_copy(pe_h.at[pl.ds(pbase, PPW)], pe_v)
    pltpu.sync_copy(te_h, te_v)
    pltpu.sync_copy(g_h, g_v)
    pltpu.sync_copy(bt_h, b_v)
    for b in range(B):
        bs = pl.ds(b * S + pbase, PPW)
        pltpu.sync_copy(ids_h.at[bs], idsw_v.at[pl.ds(b * PPW, PPW)])
        pltpu.sync_copy(tts_h.at[bs], ttw_v.at[pl.ds(b * PPW, PPW)])
    @plsc.parallel_loop(0, B * PPW // L, unroll=2)
    def ttconv(q):
        qs = pl.ds(q * L, L)
        ttfw_v[qs] = ttw_v[qs].astype(jnp.float32)

    for h in range(NH):
        hs = pl.ds(h * L, L)
        te_v[1, hs] = te_v[1, hs] - te_v[0, hs]

    @plsc.parallel_loop(0, PPW, unroll=2)
    def fold_body(t):
        for h in range(NH):
            hs = pl.ds(h * L, L)
            pe_v[t, hs] = pe_v[t, hs] + te_v[0, hs]

    inv_h = jnp.float32(1.0 / HID)

    pltpu.async_copy(we_h.at[idsw_v.at[pl.ds(0, C)]], gb0, sg0)

    def pair_body(kk, _):
        for par in range(2):
            k = kk * 2 + par
            tbase, poff = tbase_of(k)
            nxt = 1 - par

            @pl.when(k < NCHUNK - 1)
            def _():
                pltpu.async_copy(we_h.at[idsw_v.at[pl.ds((k + 1) * C, C)]],
                                 gbs[nxt], sgs[nxt])

            pltpu.make_async_copy(we_h.at[idsw_v.at[pl.ds(k * C, C)]],
                                  gbs[par], sgs[par]).wait()

            fgroup = ttfw_v[pl.ds(k * C, L)]
            rows_v = gbs[par]
            out_v = obs[par]

            def block_body(tb, _):
                t0 = tb * TB
                lane0 = t0
                fv = [
                    fgroup.at[jnp.full((L,), lane0 + j, jnp.int32)].get(
                        mode="promise_in_bounds")
                    for j in range(TB)
                ]
                zero = jnp.zeros((L,), jnp.float32)

                @plsc.parallel_loop(0, NH, carry=tuple([zero] * (2 * TB)))
                def moments(h, acc):
                    hs = pl.ds(h * L, L)
                    d = te_v[1, hs]
                    out = []
                    for j in range(TB):
                        e = (rows_v[t0 + j, hs] + pe_v[poff + t0 + j, hs]
                             + fv[j] * d)
                        ebuf_v[t0 + j, hs] = e
                        out.append(acc[2 * j] + e)
                        out.append(acc[2 * j + 1] + e * e)
                    return tuple(out)

                aa = []
                bb = []
                for j in range(TB):
                    mean = _lane_sum(moments[2 * j]) * inv_h
                    var = (_lane_sum(moments[2 * j + 1]) * inv_h
                           - mean * mean)
                    rstd = _rsqrt(var + jnp.float32(EPS))
                    aa.append(rstd)
                    bb.append(-mean * rstd)

                @plsc.parallel_loop(0, NH)
                def normalize(h):
                    hs = pl.ds(h * L, L)
                    g = g_v[hs]
                    bt = b_v[hs]
                    for j in range(TB):
                        e = ebuf_v[t0 + j, hs]
                        out_v[t0 + j, hs] = (e * aa[j] + bb[j]) * g + bt
                return 0

            @pl.when(k >= 2)
            def _():
                ptbase, _ = tbase_of(k - 2)
                pltpu.make_async_copy(obs[par],
                                      out_h.at[pl.ds(ptbase, C)],
                                      sos[par]).wait()

            lax.fori_loop(0, C // TB, block_body, 0)
            pltpu.async_copy(obs[par], out_h.at[pl.ds(tbase, C)], sos[par])
        return 0

    lax.fori_loop(0, NCHUNK // 2, pair_body, 0)

    for k in (NCHUNK - 2, NCHUNK - 1):
        par = k % 2
        tbase, _ = tbase_of(k)
        pltpu.make_async_copy(obs[par], out_h.at[pl.ds(tbase, C)],
                              sos[par]).wait()


def kernel(input_ids, token_type_ids, word_emb, pos_emb, type_emb, gamma, beta):
    ids = input_ids.reshape(-1).astype(jnp.int32)
    tts = token_type_ids.reshape(-1).astype(jnp.int32)
    out = _emb_kernel(ids, tts, word_emb, pos_emb, type_emb, gamma, beta)
    return out.reshape(B, S, HID)


# speedup vs baseline: 2.8181x; 1.0157x over previous
import functools

import jax
import jax.numpy as jnp
from jax import lax
from jax.experimental import pallas as pl
from jax.experimental.pallas import tpu as pltpu
from jax.experimental.pallas import tpu_sc as plsc

HID = 768
L = 16
NH = HID // L
NC = 2
NS = 16
NW = NC * NS
B = 4
S = 2048
PPW = S // NW
C = 16
CPB = PPW // C
NCHUNK = B * CPB
TB = 8
EPS = 1e-12


def _rsqrt(xv):
    yi = lax.bitcast_convert_type(xv, jnp.int32)
    y = lax.bitcast_convert_type(
        jnp.int32(0x5F3759DF) - lax.shift_right_logical(yi, 1), jnp.float32)
    for _ in range(3):
        y = y * (1.5 - 0.5 * xv * y * y)
    return y


def _lane_sum(v):
    for sh in (8, 4, 2, 1):
        perm = lax.broadcasted_iota(jnp.int32, (L,), 0) ^ sh
        v = v + v.at[perm].get(mode="promise_in_bounds")
    return v


mesh = plsc.VectorSubcoreMesh(core_axis_name="c", subcore_axis_name="s")


@functools.partial(
    pl.kernel,
    mesh=mesh,
    out_type=jax.ShapeDtypeStruct((B * S, HID), jnp.float32),
    compiler_params=pltpu.CompilerParams(needs_layout_passes=False),
    scratch_types=[
        pltpu.VMEM((B * PPW,), jnp.int32),
        pltpu.VMEM((B * PPW,), jnp.int32),
        pltpu.VMEM((B * PPW,), jnp.float32),
        pltpu.VMEM((C, HID), jnp.float32),
        pltpu.VMEM((C, HID), jnp.float32),
        pltpu.VMEM((C, HID), jnp.float32),
        pltpu.VMEM((C, HID), jnp.float32),
        pltpu.VMEM((PPW, HID), jnp.float32),
        pltpu.VMEM((2, HID), jnp.float32),
        pltpu.VMEM((HID,), jnp.float32),
        pltpu.VMEM((HID,), jnp.float32),
        pltpu.SemaphoreType.DMA,
        pltpu.SemaphoreType.DMA,
        pltpu.SemaphoreType.DMA,
        pltpu.SemaphoreType.DMA,
    ],
)
def _emb_kernel(ids_h, tts_h, we_h, pe_h, te_h, g_h, bt_h, out_h,
                idsw_v, ttw_v, ttfw_v, gb0, gb1, ob0, ob1,
                pe_v, te_v, g_v, b_v, sg0, sg1, so0, so1):
    wid = lax.axis_index("s") * NC + lax.axis_index("c")
    pbase = wid * PPW
    gbs = (gb0, gb1)
    obs = (ob0, ob1)
    sgs = (sg0, sg1)
    sos = (so0, so1)

    def tbase_of(k):
        b = k // CPB
        half = k % CPB
        return b * S + pbase + half * C, half * C

    pltpu.sync_copy(pe_h.at[pl.ds(pbase, PPW)], pe_v)
    pltpu.sync_copy(te_h, te_v)
    pltpu.sync_copy(g_h, g_v)
    pltpu.sync_copy(bt_h, b_v)
    for b in range(B):
        bs = pl.ds(b * S + pbase, PPW)
        pltpu.sync_copy(ids_h.at[bs], idsw_v.at[pl.ds(b * PPW, PPW)])
        pltpu.sync_copy(tts_h.at[bs], ttw_v.at[pl.ds(b * PPW, PPW)])
    @plsc.parallel_loop(0, B * PPW // L, unroll=2)
    def ttconv(q):
        qs = pl.ds(q * L, L)
        ttfw_v[qs] = ttw_v[qs].astype(jnp.float32)

    for h in range(NH):
        hs = pl.ds(h * L, L)
        te_v[1, hs] = te_v[1, hs] - te_v[0, hs]

    @plsc.parallel_loop(0, PPW, unroll=2)
    def fold_body(t):
        for h in range(NH):
            hs = pl.ds(h * L, L)
            pe_v[t, hs] = pe_v[t, hs] + te_v[0, hs]

    inv_h = jnp.float32(1.0 / HID)

    pltpu.async_copy(we_h.at[idsw_v.at[pl.ds(0, C)]], gb0, sg0)

    def pair_body(kk, _):
        for par in range(2):
            k = kk * 2 + par
            tbase, poff = tbase_of(k)
            nxt = 1 - par

            @pl.when(k < NCHUNK - 1)
            def _():
                pltpu.async_copy(we_h.at[idsw_v.at[pl.ds((k + 1) * C, C)]],
                                 gbs[nxt], sgs[nxt])

            pltpu.make_async_copy(we_h.at[idsw_v.at[pl.ds(k * C, C)]],
                                  gbs[par], sgs[par]).wait()

            fgroup = ttfw_v[pl.ds(k * C, L)]
            rows_v = gbs[par]
            out_v = obs[par]

            def block_body(tb, _):
                t0 = tb * TB
                lane0 = t0
                fv = [
                    fgroup.at[jnp.full((L,), lane0 + j, jnp.int32)].get(
                        mode="promise_in_bounds")
                    for j in range(TB)
                ]
                zero = jnp.zeros((L,), jnp.float32)

                @plsc.parallel_loop(0, NH, carry=tuple([zero] * (2 * TB)))
                def moments(h, acc):
                    hs = pl.ds(h * L, L)
                    d = te_v[1, hs]
                    out = []
                    for j in range(TB):
                        e = (rows_v[t0 + j, hs] + pe_v[poff + t0 + j, hs]
                             + fv[j] * d)
                        out.append(acc[2 * j] + e)
                        out.append(acc[2 * j + 1] + e * e)
                    return tuple(out)

                aa = []
                bb = []
                for j in range(TB):
                    mean = _lane_sum(moments[2 * j]) * inv_h
                    var = (_lane_sum(moments[2 * j + 1]) * inv_h
                           - mean * mean)
                    rstd = _rsqrt(var + jnp.float32(EPS))
                    aa.append(rstd)
                    bb.append(-mean * rstd)

                @plsc.parallel_loop(0, NH)
                def normalize(h):
                    hs = pl.ds(h * L, L)
                    d = te_v[1, hs]
                    g = g_v[hs]
                    bt = b_v[hs]
                    for j in range(TB):
                        e = (rows_v[t0 + j, hs] + pe_v[poff + t0 + j, hs]
                             + fv[j] * d)
                        out_v[t0 + j, hs] = (e * aa[j] + bb[j]) * g + bt
                return 0

            @pl.when(k >= 2)
            def _():
                ptbase, _ = tbase_of(k - 2)
                pltpu.make_async_copy(obs[par],
                                      out_h.at[pl.ds(ptbase, C)],
                                      sos[par]).wait()

            lax.fori_loop(0, C // TB, block_body, 0)
            pltpu.async_copy(obs[par], out_h.at[pl.ds(tbase, C)], sos[par])
        return 0

    lax.fori_loop(0, NCHUNK // 2, pair_body, 0)

    for k in (NCHUNK - 2, NCHUNK - 1):
        par = k % 2
        tbase, _ = tbase_of(k)
        pltpu.make_async_copy(obs[par], out_h.at[pl.ds(tbase, C)],
                              sos[par]).wait()


def kernel(input_ids, token_type_ids, word_emb, pos_emb, type_emb, gamma, beta):
    ids = input_ids.reshape(-1).astype(jnp.int32)
    tts = token_type_ids.reshape(-1).astype(jnp.int32)
    out = _emb_kernel(ids, tts, word_emb, pos_emb, type_emb, gamma, beta)
    return out.reshape(B, S, HID)


# speedup vs baseline: 3.2650x; 1.1586x over previous
import functools

import jax
import jax.numpy as jnp
from jax import lax
from jax.experimental import pallas as pl
from jax.experimental.pallas import tpu as pltpu
from jax.experimental.pallas import tpu_sc as plsc

HID = 768
L = 16
NH = HID // L
NC = 2
NS = 16
NW = NC * NS
B = 4
S = 2048
PPW = S // NW
C = 16
CPB = PPW // C
NCHUNK = B * CPB
TB = 8
EPS = 1e-12


def _rsqrt(xv):
    yi = lax.bitcast_convert_type(xv, jnp.int32)
    y = lax.bitcast_convert_type(
        jnp.int32(0x5F3759DF) - lax.shift_right_logical(yi, 1), jnp.float32)
    for _ in range(3):
        y = y * (1.5 - 0.5 * xv * y * y)
    return y


def _lane_sum(v):
    for sh in (8, 4, 2, 1):
        perm = lax.broadcasted_iota(jnp.int32, (L,), 0) ^ sh
        v = v + v.at[perm].get(mode="promise_in_bounds")
    return v


mesh = plsc.VectorSubcoreMesh(core_axis_name="c", subcore_axis_name="s")


@functools.partial(
    pl.kernel,
    mesh=mesh,
    out_type=jax.ShapeDtypeStruct((B * S, HID), jnp.float32),
    compiler_params=pltpu.CompilerParams(needs_layout_passes=False),
    scratch_types=[
        pltpu.VMEM((B * PPW,), jnp.int32),
        pltpu.VMEM((B * PPW,), jnp.int32),
        pltpu.VMEM((B * PPW,), jnp.float32),
        pltpu.VMEM((C, HID), jnp.float32),
        pltpu.VMEM((C, HID), jnp.float32),
        pltpu.VMEM((C, HID), jnp.float32),
        pltpu.VMEM((C, HID), jnp.float32),
        pltpu.VMEM((C, HID), jnp.float32),
        pltpu.VMEM((PPW, HID), jnp.float32),
        pltpu.VMEM((2, HID), jnp.float32),
        pltpu.VMEM((HID,), jnp.float32),
        pltpu.VMEM((HID,), jnp.float32),
        pltpu.SemaphoreType.DMA,
        pltpu.SemaphoreType.DMA,
        pltpu.SemaphoreType.DMA,
        pltpu.SemaphoreType.DMA,
    ],
)
def _emb_kernel(ids_h, tts_h, we_h, pe_h, te_h, g_h, bt_h, out_h,
                idsw_v, ttw_v, ttfw_v, gb0, gb1, ebuf_v, ob0, ob1,
                pe_v, te_v, g_v, b_v, sg0, sg1, so0, so1):
    wid = lax.axis_index("s") * NC + lax.axis_index("c")
    pbase = wid * PPW
    gbs = (gb0, gb1)
    obs = (ob0, ob1)
    sgs = (sg0, sg1)
    sos = (so0, so1)

    def tbase_of(k):
        b = k // CPB
        half = k % CPB
        return b * S + pbase + half * C, half * C

    pltpu.sync_copy(pe_h.at[pl.ds(pbase, PPW)], pe_v)
    pltpu.sync_copy(te_h, te_v)
    pltpu.sync_copy(g_h, g_v)
    pltpu.sync_copy(bt_h, b_v)
    for b in range(B):
        bs = pl.ds(b * S + pbase, PPW)
        pltpu.sync_copy(ids_h.at[bs], idsw_v.at[pl.ds(b * PPW, PPW)])
        pltpu.sync_copy(tts_h.at[bs], ttw_v.at[pl.ds(b * PPW, PPW)])
    @plsc.parallel_loop(0, B * PPW // L, unroll=2)
    def ttconv(q):
        qs = pl.ds(q * L, L)
        ttfw_v[qs] = ttw_v[qs].astype(jnp.float32)

    for h in range(NH):
        hs = pl.ds(h * L, L)
        te_v[1, hs] = te_v[1, hs] - te_v[0, hs]

    @plsc.parallel_loop(0, PPW, unroll=2)
    def fold_body(t):
        for h in range(NH):
            hs = pl.ds(h * L, L)
            pe_v[t, hs] = pe_v[t, hs] + te_v[0, hs]

    inv_h = jnp.float32(1.0 / HID)

    pltpu.async_copy(we_h.at[idsw_v.at[pl.ds(0, C)]], gb0, sg0)

    def pair_body(kk, _):
        for par in range(2):
            k = kk * 2 + par
            tbase, poff = tbase_of(k)
            nxt = 1 - par

            @pl.when(k < NCHUNK - 1)
            def _():
                pltpu.async_copy(we_h.at[idsw_v.at[pl.ds((k + 1) * C, C)]],
                                 gbs[nxt], sgs[nxt])

            pltpu.make_async_copy(we_h.at[idsw_v.at[pl.ds(k * C, C)]],
                                  gbs[par], sgs[par]).wait()

            fgroup = ttfw_v[pl.ds(k * C, L)]
            rows_v = gbs[par]
            out_v = obs[par]

            def block_body(tb, _):
                t0 = tb * TB
                lane0 = t0
                fv = [
                    fgroup.at[jnp.full((L,), lane0 + j, jnp.int32)].get(
                        mode="promise_in_bounds")
                    for j in range(TB)
                ]
                zero = jnp.zeros((L,), jnp.float32)

                @plsc.parallel_loop(0, NH, carry=tuple([zero] * (2 * TB)))
                def moments(h, acc):
                    hs = pl.ds(h * L, L)
                    d = te_v[1, hs]
                    out = []
                    for j in range(TB):
                        e = (rows_v[t0 + j, hs] + pe_v[poff + t0 + j, hs]
                             + fv[j] * d)
                        ebuf_v[t0 + j, hs] = e
                        out.append(acc[2 * j] + e)
                        out.append(acc[2 * j + 1] + e * e)
                    return tuple(out)

                aa = []
                bb = []
                for j in range(TB):
                    mean = _lane_sum(moments[2 * j]) * inv_h
                    var = (_lane_sum(moments[2 * j + 1]) * inv_h
                           - mean * mean)
                    rstd = _rsqrt(var + jnp.float32(EPS))
                    aa.append(rstd)
                    bb.append(-mean * rstd)

                @plsc.parallel_loop(0, NH)
                def normalize(h):
                    hs = pl.ds(h * L, L)
                    g = g_v[hs]
                    bt = b_v[hs]
                    for j in range(TB):
                        e = ebuf_v[t0 + j, hs]
                        out_v[t0 + j, hs] = (e * aa[j] + bb[j]) * g + bt
                return 0

            @pl.when(k >= 2)
            def _():
                ptbase, _ = tbase_of(k - 2)
                pltpu.make_async_copy(obs[par],
                                      out_h.at[pl.ds(ptbase, C)],
                                      sos[par]).wait()

            lax.fori_loop(0, C // TB, block_body, 0)
            pltpu.async_copy(obs[par], out_h.at[pl.ds(tbase, C)], sos[par])
        return 0

    lax.fori_loop(0, NCHUNK // 2, pair_body, 0)

    for k in (NCHUNK - 2, NCHUNK - 1):
        par = k % 2
        tbase, _ = tbase_of(k)
        pltpu.make_async_copy(obs[par], out_h.at[pl.ds(tbase, C)],
                              sos[par]).wait()


def kernel(input_ids, token_type_ids, word_emb, pos_emb, type_emb, gamma, beta):
    ids = input_ids.reshape(-1).astype(jnp.int32)
    tts = token_type_ids.reshape(-1).astype(jnp.int32)
    out = _emb_kernel(ids, tts, word_emb, pos_emb, type_emb, gamma, beta)
    return out.reshape(B, S, HID)


# speedup vs baseline: 3.5710x; 1.0937x over previous
import functools

import jax
import jax.numpy as jnp
from jax import lax
from jax.experimental import pallas as pl
from jax.experimental.pallas import tpu as pltpu
from jax.experimental.pallas import tpu_sc as plsc

HID = 768
L = 16
NH = HID // L
NC = 2
NS = 16
NW = NC * NS
B = 4
S = 2048
PPW = S // NW
C = 16
CPB = PPW // C
NCHUNK = B * CPB
TB = 8
EPS = 1e-12


def _rsqrt(xv):
    yi = lax.bitcast_convert_type(xv, jnp.int32)
    y = lax.bitcast_convert_type(
        jnp.int32(0x5F3759DF) - lax.shift_right_logical(yi, 1), jnp.float32)
    for _ in range(3):
        y = y * (1.5 - 0.5 * xv * y * y)
    return y


def _lane_sum(v):
    for sh in (8, 4, 2, 1):
        perm = lax.broadcasted_iota(jnp.int32, (L,), 0) ^ sh
        v = v + v.at[perm].get(mode="promise_in_bounds")
    return v


mesh = plsc.VectorSubcoreMesh(core_axis_name="c", subcore_axis_name="s")


@functools.partial(
    pl.kernel,
    mesh=mesh,
    out_type=jax.ShapeDtypeStruct((B * S, HID), jnp.float32),
    compiler_params=pltpu.CompilerParams(needs_layout_passes=False),
    scratch_types=[
        pltpu.VMEM((B * PPW,), jnp.int32),
        pltpu.VMEM((B * PPW,), jnp.int32),
        pltpu.VMEM((B * PPW,), jnp.float32),
        pltpu.VMEM((C, HID), jnp.float32),
        pltpu.VMEM((C, HID), jnp.float32),
        pltpu.VMEM((C, HID), jnp.float32),
        pltpu.VMEM((C, HID), jnp.float32),
        pltpu.VMEM((C, HID), jnp.float32),
        pltpu.VMEM((PPW, HID), jnp.float32),
        pltpu.VMEM((2, HID), jnp.float32),
        pltpu.VMEM((HID,), jnp.float32),
        pltpu.VMEM((HID,), jnp.float32),
        pltpu.SemaphoreType.DMA,
        pltpu.SemaphoreType.DMA,
        pltpu.SemaphoreType.DMA,
        pltpu.SemaphoreType.DMA,
    ],
)
def _emb_kernel(ids_h, tts_h, we_h, pe_h, te_h, g_h, bt_h, out_h,
                idsw_v, ttw_v, ttfw_v, gb0, gb1, ebuf_v, ob0, ob1,
                pe_v, te_v, g_v, b_v, sg0, sg1, so0, so1):
    wid = lax.axis_index("s") * NC + lax.axis_index("c")
    pbase = wid * PPW
    gbs = (gb0, gb1)
    obs = (ob0, ob1)
    sgs = (sg0, sg1)
    sos = (so0, so1)

    def tbase_of(k):
        b = k // CPB
        half = k % CPB
        return b * S + pbase + half * C, half * C

    pe_cp = pltpu.make_async_copy(pe_h.at[pl.ds(pbase, PPW)], pe_v, sg1)
    pe_cp.start()
    id_cps = []
    for b in range(B):
        bs = pl.ds(b * S + pbase, PPW)
        c1 = pltpu.make_async_copy(ids_h.at[bs],
                                   idsw_v.at[pl.ds(b * PPW, PPW)], so0)
        c2 = pltpu.make_async_copy(tts_h.at[bs],
                                   ttw_v.at[pl.ds(b * PPW, PPW)], so1)
        c1.start()
        c2.start()
        id_cps += [c1, c2]
    pltpu.sync_copy(te_h, te_v)
    pltpu.sync_copy(g_h, g_v)
    pltpu.sync_copy(bt_h, b_v)

    for h in range(NH):
        hs = pl.ds(h * L, L)
        te_v[1, hs] = te_v[1, hs] - te_v[0, hs]

    for cp in id_cps:
        cp.wait()

    pltpu.async_copy(we_h.at[idsw_v.at[pl.ds(0, C)]], gb0, sg0)

    @plsc.parallel_loop(0, B * PPW // L, unroll=2)
    def ttconv(q):
        qs = pl.ds(q * L, L)
        ttfw_v[qs] = ttw_v[qs].astype(jnp.float32)

    pe_cp.wait()

    @plsc.parallel_loop(0, PPW, unroll=2)
    def fold_body(t):
        for h in range(NH):
            hs = pl.ds(h * L, L)
            pe_v[t, hs] = pe_v[t, hs] + te_v[0, hs]

    inv_h = jnp.float32(1.0 / HID)

    def pair_body(kk, _):
        for par in range(2):
            k = kk * 2 + par
            tbase, poff = tbase_of(k)
            nxt = 1 - par

            @pl.when(k < NCHUNK - 1)
            def _():
                pltpu.async_copy(we_h.at[idsw_v.at[pl.ds((k + 1) * C, C)]],
                                 gbs[nxt], sgs[nxt])

            pltpu.make_async_copy(we_h.at[idsw_v.at[pl.ds(k * C, C)]],
                                  gbs[par], sgs[par]).wait()

            fgroup = ttfw_v[pl.ds(k * C, L)]
            rows_v = gbs[par]
            out_v = obs[par]

            def block_body(tb, _):
                t0 = tb * TB
                lane0 = t0
                fv = [
                    fgroup.at[jnp.full((L,), lane0 + j, jnp.int32)].get(
                        mode="promise_in_bounds")
                    for j in range(TB)
                ]
                zero = jnp.zeros((L,), jnp.float32)

                @plsc.parallel_loop(0, NH, carry=tuple([zero] * (2 * TB)))
                def moments(h, acc):
                    hs = pl.ds(h * L, L)
                    d = te_v[1, hs]
                    out = []
                    for j in range(TB):
                        e = (rows_v[t0 + j, hs] + pe_v[poff + t0 + j, hs]
                             + fv[j] * d)
                        ebuf_v[t0 + j, hs] = e
                        out.append(acc[2 * j] + e)
                        out.append(acc[2 * j + 1] + e * e)
                    return tuple(out)

                aa = []
                bb = []
                for j in range(TB):
                    mean = _lane_sum(moments[2 * j]) * inv_h
                    var = (_lane_sum(moments[2 * j + 1]) * inv_h
                           - mean * mean)
                    rstd = _rsqrt(var + jnp.float32(EPS))
                    aa.append(rstd)
                    bb.append(-mean * rstd)

                @plsc.parallel_loop(0, NH)
                def normalize(h):
                    hs = pl.ds(h * L, L)
                    g = g_v[hs]
                    bt = b_v[hs]
                    for j in range(TB):
                        e = ebuf_v[t0 + j, hs]
                        out_v[t0 + j, hs] = (e * aa[j] + bb[j]) * g + bt
                return 0

            @pl.when(k >= 2)
            def _():
                ptbase, _ = tbase_of(k - 2)
                pltpu.make_async_copy(obs[par],
                                      out_h.at[pl.ds(ptbase, C)],
                                      sos[par]).wait()

            lax.fori_loop(0, C // TB, block_body, 0)
            pltpu.async_copy(obs[par], out_h.at[pl.ds(tbase, C)], sos[par])
        return 0

    lax.fori_loop(0, NCHUNK // 2, pair_body, 0)

    for k in (NCHUNK - 2, NCHUNK - 1):
        par = k % 2
        tbase, _ = tbase_of(k)
        pltpu.make_async_copy(obs[par], out_h.at[pl.ds(tbase, C)],
                              sos[par]).wait()


def kernel(input_ids, token_type_ids, word_emb, pos_emb, type_emb, gamma, beta):
    ids = input_ids.reshape(-1).astype(jnp.int32)
    tts = token_type_ids.reshape(-1).astype(jnp.int32)
    out = _emb_kernel(ids, tts, word_emb, pos_emb, type_emb, gamma, beta)
    return out.reshape(B, S, HID)


# speedup vs baseline: 3.5937x; 1.0064x over previous
import functools

import jax
import jax.numpy as jnp
from jax import lax
from jax.experimental import pallas as pl
from jax.experimental.pallas import tpu as pltpu
from jax.experimental.pallas import tpu_sc as plsc

HID = 768
L = 16
NH = HID // L
NC = 2
NS = 16
NW = NC * NS
B = 4
S = 2048
PPW = S // NW
C = 16
CPB = PPW // C
NCHUNK = B * CPB
TB = 8
EPS = 1e-12


def _rsqrt(xv):
    yi = lax.bitcast_convert_type(xv, jnp.int32)
    y = lax.bitcast_convert_type(
        jnp.int32(0x5F3759DF) - lax.shift_right_logical(yi, 1), jnp.float32)
    for _ in range(3):
        y = y * (1.5 - 0.5 * xv * y * y)
    return y


def _lane_sum(v):
    for sh in (8, 4, 2, 1):
        perm = lax.broadcasted_iota(jnp.int32, (L,), 0) ^ sh
        v = v + v.at[perm].get(mode="promise_in_bounds")
    return v


mesh = plsc.VectorSubcoreMesh(core_axis_name="c", subcore_axis_name="s")


@functools.partial(
    pl.kernel,
    mesh=mesh,
    out_type=jax.ShapeDtypeStruct((B * S, HID), jnp.float32),
    compiler_params=pltpu.CompilerParams(needs_layout_passes=False),
    scratch_types=[
        pltpu.VMEM((B * PPW,), jnp.int32),
        pltpu.VMEM((B * PPW,), jnp.int32),
        pltpu.VMEM((B * PPW,), jnp.float32),
        pltpu.VMEM((C, HID), jnp.float32),
        pltpu.VMEM((C, HID), jnp.float32),
        pltpu.VMEM((C, HID), jnp.float32),
        pltpu.VMEM((C, HID), jnp.float32),
        pltpu.VMEM((C, HID), jnp.float32),
        pltpu.VMEM((PPW, HID), jnp.float32),
        pltpu.VMEM((2, HID), jnp.float32),
        pltpu.VMEM((HID,), jnp.float32),
        pltpu.VMEM((HID,), jnp.float32),
        pltpu.SemaphoreType.DMA,
        pltpu.SemaphoreType.DMA,
        pltpu.SemaphoreType.DMA,
        pltpu.SemaphoreType.DMA,
    ],
)
def _emb_kernel(ids_h, tts_h, we_h, pe_h, te_h, g_h, bt_h, out_h,
                idsw_v, ttw_v, ttfw_v, gb0, gb1, ebuf_v, ob0, ob1,
                pe_v, te_v, g_v, b_v, sg0, sg1, so0, so1):
    wid = lax.axis_index("s") * NC + lax.axis_index("c")
    pbase = wid * PPW
    gbs = (gb0, gb1)
    obs = (ob0, ob1)
    sgs = (sg0, sg1)
    sos = (so0, so1)

    def tbase_of(k):
        b = k // CPB
        half = k % CPB
        return b * S + pbase + half * C, half * C

    pe_cp = pltpu.make_async_copy(pe_h.at[pl.ds(pbase, PPW)], pe_v, sg1)
    pe_cp.start()
    id_cps = []
    for b in range(B):
        bs = pl.ds(b * S + pbase, PPW)
        c1 = pltpu.make_async_copy(ids_h.at[bs],
                                   idsw_v.at[pl.ds(b * PPW, PPW)], so0)
        c2 = pltpu.make_async_copy(tts_h.at[bs],
                                   ttw_v.at[pl.ds(b * PPW, PPW)], so1)
        c1.start()
        c2.start()
        id_cps += [c1, c2]
    pltpu.sync_copy(te_h, te_v)
    pltpu.sync_copy(g_h, g_v)
    pltpu.sync_copy(bt_h, b_v)

    for h in range(NH):
        hs = pl.ds(h * L, L)
        te_v[1, hs] = te_v[1, hs] - te_v[0, hs]

    for cp in id_cps:
        cp.wait()

    pltpu.async_copy(we_h.at[idsw_v.at[pl.ds(0, C)]], gb0, sg0)

    @plsc.parallel_loop(0, B * PPW // L, unroll=2)
    def ttconv(q):
        qs = pl.ds(q * L, L)
        ttfw_v[qs] = ttw_v[qs].astype(jnp.float32)

    pe_cp.wait()

    @plsc.parallel_loop(0, PPW, unroll=2)
    def fold_body(t):
        for h in range(NH):
            hs = pl.ds(h * L, L)
            pe_v[t, hs] = pe_v[t, hs] + te_v[0, hs]

    inv_h = jnp.float32(1.0 / HID)

    def pair_body(kk, _):
        for par in range(2):
            k = kk * 2 + par
            tbase, poff = tbase_of(k)
            nxt = 1 - par

            @pl.when(k < NCHUNK - 1)
            def _():
                pltpu.async_copy(we_h.at[idsw_v.at[pl.ds((k + 1) * C, C)]],
                                 gbs[nxt], sgs[nxt])

            pltpu.make_async_copy(we_h.at[idsw_v.at[pl.ds(k * C, C)]],
                                  gbs[par], sgs[par]).wait()

            fgroup = ttfw_v[pl.ds(k * C, L)]
            rows_v = gbs[par]
            out_v = obs[par]

            def block_body(tb, _):
                t0 = tb * TB
                lane0 = t0
                fv = [
                    fgroup.at[jnp.full((L,), lane0 + j, jnp.int32)].get(
                        mode="promise_in_bounds")
                    for j in range(TB)
                ]
                zero = jnp.zeros((L,), jnp.float32)

                @plsc.parallel_loop(0, NH, carry=tuple([zero] * (2 * TB)),
                                    unroll=2)
                def moments(h, acc):
                    hs = pl.ds(h * L, L)
                    d = te_v[1, hs]
                    out = []
                    for j in range(TB):
                        e = (rows_v[t0 + j, hs] + pe_v[poff + t0 + j, hs]
                             + fv[j] * d)
                        ebuf_v[t0 + j, hs] = e
                        out.append(acc[2 * j] + e)
                        out.append(acc[2 * j + 1] + e * e)
                    return tuple(out)

                aa = []
                bb = []
                for j in range(TB):
                    mean = _lane_sum(moments[2 * j]) * inv_h
                    var = (_lane_sum(moments[2 * j + 1]) * inv_h
                           - mean * mean)
                    rstd = _rsqrt(var + jnp.float32(EPS))
                    aa.append(rstd)
                    bb.append(-mean * rstd)

                @plsc.parallel_loop(0, NH, unroll=2)
                def normalize(h):
                    hs = pl.ds(h * L, L)
                    g = g_v[hs]
                    bt = b_v[hs]
                    for j in range(TB):
                        e = ebuf_v[t0 + j, hs]
                        out_v[t0 + j, hs] = (e * aa[j] + bb[j]) * g + bt
                return 0

            @pl.when(k >= 2)
            def _():
                ptbase, _ = tbase_of(k - 2)
                pltpu.make_async_copy(obs[par],
                                      out_h.at[pl.ds(ptbase, C)],
                                      sos[par]).wait()

            lax.fori_loop(0, C // TB, block_body, 0)
            pltpu.async_copy(obs[par], out_h.at[pl.ds(tbase, C)], sos[par])
        return 0

    lax.fori_loop(0, NCHUNK // 2, pair_body, 0)

    for k in (NCHUNK - 2, NCHUNK - 1):
        par = k % 2
        tbase, _ = tbase_of(k)
        pltpu.make_async_copy(obs[par], out_h.at[pl.ds(tbase, C)],
                              sos[par]).wait()


def kernel(input_ids, token_type_ids, word_emb, pos_emb, type_emb, gamma, beta):
    ids = input_ids.reshape(-1).astype(jnp.int32)
    tts = token_type_ids.reshape(-1).astype(jnp.int32)
    out = _emb_kernel(ids, tts, word_emb, pos_emb, type_emb, gamma, beta)
    return out.reshape(B, S, HID)
